# baseline (speedup 1.0000x reference)
import functools

import jax
import jax.numpy as jnp
from jax import lax
from jax.experimental import pallas as pl
from jax.experimental.pallas import tpu as pltpu
from jax.experimental.pallas import tpu_sc as plsc

T = 2048
C = 1024
E = 16
H = 2048
KTOP = 2
TILE = 128
NTILES = T * KTOP // TILE + E
TMAPN = ((NTILES + 7) // 8) * 8
NPAD = NTILES * TILE
NW = 32
TOKW = T // NW
CH = TOKW // 4


def _router_body(x_ref, wg_ref, rb_ref,
                 d0_ref, d1_ref, g0_ref, g1_ref, tmap_ref, aux_ref):
    x = x_ref[...]
    wg = wg_ref[...]
    logits = lax.dot_general(x, wg, (((1,), (1,)), ((), ())),
                             preferred_element_type=jnp.float32)
    logits = logits + rb_ref[...]
    m = jnp.max(logits, axis=1, keepdims=True)
    p = jnp.exp(logits - m)
    gates = p / jnp.sum(p, axis=1, keepdims=True)

    imp = jnp.sum(gates, axis=0, keepdims=True) * (1.0 / T)
    aux_ref[...] = float(E) * jnp.sum(imp * imp, axis=1, keepdims=True)

    lane = lax.broadcasted_iota(jnp.int32, (T, E), 1).astype(jnp.float32)
    v1 = jnp.max(gates, axis=1, keepdims=True)
    i1 = jnp.min(jnp.where(gates == v1, lane, 1e9), axis=1, keepdims=True)
    masked = jnp.where(lane == i1, -1.0, gates)
    v2 = jnp.max(masked, axis=1, keepdims=True)
    i2 = jnp.min(jnp.where(masked == v2, lane, 1e9), axis=1, keepdims=True)
    oh1 = (lane == i1).astype(jnp.float32)
    oh2 = (lane == i2).astype(jnp.float32)

    r_i = lax.broadcasted_iota(jnp.int32, (T, T), 0)
    c_i = lax.broadcasted_iota(jnp.int32, (T, T), 1)
    tri = (c_i <= r_i).astype(jnp.bfloat16)
    incl1 = lax.dot_general(tri, oh1.astype(jnp.bfloat16),
                            (((1,), (0,)), ((), ())),
                            preferred_element_type=jnp.float32)
    incl2 = lax.dot_general(tri, oh2.astype(jnp.bfloat16),
                            (((1,), (0,)), ((), ())),
                            preferred_element_type=jnp.float32)
    cnt1 = jnp.sum(oh1, axis=0, keepdims=True)
    cnt2 = jnp.sum(oh2, axis=0, keepdims=True)
    rank1 = jnp.sum((incl1 - oh1) * oh1, axis=1, keepdims=True)
    rank2 = jnp.sum((incl2 - oh2 + cnt1) * oh2, axis=1, keepdims=True)

    counts = cnt1 + cnt2
    padded = jnp.floor((counts + float(TILE - 1)) * (1.0 / TILE)) * float(TILE)
    re = lax.broadcasted_iota(jnp.int32, (E, E), 0)
    ce = lax.broadcasted_iota(jnp.int32, (E, E), 1)
    tri_e = (re < ce).astype(jnp.float32)
    offs = lax.dot_general(padded, tri_e, (((1,), (0,)), ((), ())),
                           preferred_element_type=jnp.float32)

    dest1 = jnp.sum(oh1 * offs, axis=1, keepdims=True) + rank1
    dest2 = jnp.sum(oh2 * offs, axis=1, keepdims=True) + rank2
    d0_ref[...] = dest1.astype(jnp.int32)
    d1_ref[...] = dest2.astype(jnp.int32)
    g0_ref[...] = jnp.broadcast_to(v1, (T, 128))
    g1_ref[...] = jnp.broadcast_to(v2, (T, 128))

    ts = lax.broadcasted_iota(jnp.int32, (TMAPN, E), 0).astype(jnp.float32) * float(TILE)
    le = (jnp.broadcast_to(offs, (TMAPN, E)) <= ts).astype(jnp.float32)
    tmap_ref[...] = (jnp.sum(le, axis=1, keepdims=True) - 1.0).astype(jnp.int32)


def _router(xf, wg, rb):
    return pl.pallas_call(
        _router_body,
        out_shape=[
            jax.ShapeDtypeStruct((T, 1), jnp.int32),
            jax.ShapeDtypeStruct((T, 1), jnp.int32),
            jax.ShapeDtypeStruct((T, 128), jnp.float32),
            jax.ShapeDtypeStruct((T, 128), jnp.float32),
            jax.ShapeDtypeStruct((TMAPN, 1), jnp.int32),
            jax.ShapeDtypeStruct((1, 1), jnp.float32),
        ],
    )(xf, wg, rb)


def _dispatch_body(x_hbm, d0_hbm, d1_hbm, xs_hbm,
                   idx0, idx1, xbuf, sem):
    c = lax.axis_index("c")
    s = lax.axis_index("s")
    wid = s * 2 + c
    t0 = wid * TOKW
    pltpu.sync_copy(d0_hbm.at[pl.ds(t0, TOKW)], idx0)
    pltpu.sync_copy(d1_hbm.at[pl.ds(t0, TOKW)], idx1)
    pltpu.sync_copy(x_hbm.at[pl.ds(t0, TOKW)], xbuf)
    cp0 = pltpu.async_copy(xbuf, xs_hbm.at[idx0], sem)
    cp1 = pltpu.async_copy(xbuf, xs_hbm.at[idx1], sem)
    cp0.wait()
    cp1.wait()


def _dispatch(xf, d0, d1):
    mesh = plsc.VectorSubcoreMesh(core_axis_name="c", subcore_axis_name="s")
    fn = functools.partial(
        pl.kernel,
        out_type=jax.ShapeDtypeStruct((NPAD, C), jnp.float32),
        mesh=mesh,
        scratch_types=[
            pltpu.VMEM((TOKW,), jnp.int32),
            pltpu.VMEM((TOKW,), jnp.int32),
            pltpu.VMEM((TOKW, C), jnp.float32),
            pltpu.SemaphoreType.DMA,
        ],
    )(_dispatch_body)
    return fn(xf, d0, d1)


def _ffn_body(tmap_ref, xs_ref, w1_ref, b1_ref, w2_ref, b2_ref, o_ref):
    xb = xs_ref[...]
    h = lax.dot_general(xb, w1_ref[0], (((1,), (0,)), ((), ())),
                        preferred_element_type=jnp.float32)
    h = h + b1_ref[0]
    h = 0.5 * h * (1.0 + lax.erf(h * 0.7071067811865476))
    o = lax.dot_general(h, w2_ref[0], (((1,), (0,)), ((), ())),
                        preferred_element_type=jnp.float32)
    o = o + b2_ref[0]
    o_ref[...] = o


def _ffn(tmap, xs, w1, b1, w2, b2):
    grid_spec = pltpu.PrefetchScalarGridSpec(
        num_scalar_prefetch=1,
        grid=(NTILES,),
        in_specs=[
            pl.BlockSpec((TILE, C), lambda t, tm: (t, 0)),
            pl.BlockSpec((1, C, H), lambda t, tm: (tm[t], 0, 0)),
            pl.BlockSpec((1, 1, H), lambda t, tm: (tm[t], 0, 0)),
            pl.BlockSpec((1, H, C), lambda t, tm: (tm[t], 0, 0)),
            pl.BlockSpec((1, 1, C), lambda t, tm: (tm[t], 0, 0)),
        ],
        out_specs=pl.BlockSpec((TILE, C), lambda t, tm: (t, 0)),
    )
    return pl.pallas_call(
        _ffn_body,
        grid_spec=grid_spec,
        out_shape=jax.ShapeDtypeStruct((NPAD, C), jnp.float32),
        compiler_params=pltpu.CompilerParams(
            dimension_semantics=("arbitrary",)),
    )(tmap, xs, w1, b1, w2, b2)


def _combine_body(os_hbm, d0_hbm, d1_hbm, g0_hbm, g1_hbm, y_hbm,
                  idx0, idx1, b0, b1, gb0, gb1, sem0, sem1):
    c = lax.axis_index("c")
    s = lax.axis_index("s")
    wid = s * 2 + c
    t0 = wid * TOKW
    nch = TOKW // CH
    sems = [sem0, sem1]

    def start(ch):
        slot = ch % 2
        base = t0 + ch * CH
        pltpu.sync_copy(d0_hbm.at[pl.ds(base, CH)], idx0.at[slot])
        pltpu.sync_copy(d1_hbm.at[pl.ds(base, CH)], idx1.at[slot])
        cp0 = pltpu.async_copy(os_hbm.at[idx0.at[slot]], b0.at[slot],
                               sems[slot])
        cp1 = pltpu.async_copy(os_hbm.at[idx1.at[slot]], b1.at[slot],
                               sems[slot])
        return cp0, cp1

    pend = [start(0), start(1)]
    for ch in range(nch):
        slot = ch % 2
        base = t0 + ch * CH
        cp0, cp1 = pend[slot]
        pltpu.sync_copy(g0_hbm.at[pl.ds(base, CH)], gb0)
        pltpu.sync_copy(g1_hbm.at[pl.ds(base, CH)], gb1)
        cp0.wait()
        cp1.wait()

        def body(r, carry):
            gv0 = gb0[r, 0:16]
            gv1 = gb1[r, 0:16]
            for cc in range(C // 16):
                sl = pl.ds(cc * 16, 16)
                b0[slot, r, sl] = b0[slot, r, sl] * gv0 + b1[slot, r, sl] * gv1
            return carry

        lax.fori_loop(0, CH, body, 0)
        pltpu.sync_copy(b0.at[slot], y_hbm.at[pl.ds(base, CH)])
        if ch + 2 < nch:
            pend[slot] = start(ch + 2)


def _combine(os, d0, d1, g0, g1):
    mesh = plsc.VectorSubcoreMesh(core_axis_name="c", subcore_axis_name="s")
    fn = functools.partial(
        pl.kernel,
        out_type=jax.ShapeDtypeStruct((T, C), jnp.float32),
        mesh=mesh,
        scratch_types=[
            pltpu.VMEM((2, CH), jnp.int32),
            pltpu.VMEM((2, CH), jnp.int32),
            pltpu.VMEM((2, CH, C), jnp.float32),
            pltpu.VMEM((2, CH, C), jnp.float32),
            pltpu.VMEM((CH, 128), jnp.float32),
            pltpu.VMEM((CH, 128), jnp.float32),
            pltpu.SemaphoreType.DMA,
            pltpu.SemaphoreType.DMA,
        ],
    )(_combine_body)
    return fn(os, d0, d1, g0, g1)


def kernel(x, w_gating, router_bias, fc1_weight, fc1_bias, fc2_weight,
           fc2_bias):
    xf = x.reshape(T, C)
    rb = router_bias.reshape(1, E)
    d0, d1, g0, g1, tmapN, aux = _router(xf, w_gating, rb)
    d0f = d0.reshape(T)
    d1f = d1.reshape(T)
    tmap = tmapN.reshape(TMAPN)[:NTILES]
    xs = _dispatch(xf, d0f, d1f)
    os_ = _ffn(tmap, xs, fc1_weight, fc1_bias.reshape(E, 1, H),
               fc2_weight, fc2_bias.reshape(E, 1, C))
    y = _combine(os_, d0f, d1f, g0, g1)
    return (y.reshape(1, T, C), aux.reshape(()))


# speedup vs baseline: 1.0242x; 1.0242x over previous
import functools

import jax
import jax.numpy as jnp
from jax import lax
from jax.experimental import pallas as pl
from jax.experimental.pallas import tpu as pltpu
from jax.experimental.pallas import tpu_sc as plsc

T = 2048
C = 1024
E = 16
H = 2048
KTOP = 2
TILE = 128
NTILES = T * KTOP // TILE + E
TMAPN = ((NTILES + 7) // 8) * 8
NPAD = NTILES * TILE
NW = 32
TOKW = T // NW
CH = TOKW // 2


def _router_body(x_ref, wg_ref, rb_ref,
                 d0_ref, d1_ref, g0_ref, g1_ref, tmap_ref, aux_ref):
    x = x_ref[...]
    wg = wg_ref[...]
    logits = lax.dot_general(x, wg, (((1,), (1,)), ((), ())),
                             preferred_element_type=jnp.float32)
    logits = logits + rb_ref[...]
    m = jnp.max(logits, axis=1, keepdims=True)
    p = jnp.exp(logits - m)
    gates = p / jnp.sum(p, axis=1, keepdims=True)

    imp = jnp.sum(gates, axis=0, keepdims=True) * (1.0 / T)
    aux_ref[...] = float(E) * jnp.sum(imp * imp, axis=1, keepdims=True)

    lane = lax.broadcasted_iota(jnp.int32, (T, E), 1).astype(jnp.float32)
    v1 = jnp.max(gates, axis=1, keepdims=True)
    i1 = jnp.min(jnp.where(gates == v1, lane, 1e9), axis=1, keepdims=True)
    masked = jnp.where(lane == i1, -1.0, gates)
    v2 = jnp.max(masked, axis=1, keepdims=True)
    i2 = jnp.min(jnp.where(masked == v2, lane, 1e9), axis=1, keepdims=True)
    oh1 = (lane == i1).astype(jnp.float32)
    oh2 = (lane == i2).astype(jnp.float32)

    r_i = lax.broadcasted_iota(jnp.int32, (T, T), 0)
    c_i = lax.broadcasted_iota(jnp.int32, (T, T), 1)
    tri = (c_i <= r_i).astype(jnp.bfloat16)
    incl1 = lax.dot_general(tri, oh1.astype(jnp.bfloat16),
                            (((1,), (0,)), ((), ())),
                            preferred_element_type=jnp.float32)
    incl2 = lax.dot_general(tri, oh2.astype(jnp.bfloat16),
                            (((1,), (0,)), ((), ())),
                            preferred_element_type=jnp.float32)
    cnt1 = jnp.sum(oh1, axis=0, keepdims=True)
    cnt2 = jnp.sum(oh2, axis=0, keepdims=True)
    rank1 = jnp.sum((incl1 - oh1) * oh1, axis=1, keepdims=True)
    rank2 = jnp.sum((incl2 - oh2 + cnt1) * oh2, axis=1, keepdims=True)

    counts = cnt1 + cnt2
    padded = jnp.floor((counts + float(TILE - 1)) * (1.0 / TILE)) * float(TILE)
    re = lax.broadcasted_iota(jnp.int32, (E, E), 0)
    ce = lax.broadcasted_iota(jnp.int32, (E, E), 1)
    tri_e = (re < ce).astype(jnp.float32)
    offs = lax.dot_general(padded, tri_e, (((1,), (0,)), ((), ())),
                           preferred_element_type=jnp.float32)

    dest1 = jnp.sum(oh1 * offs, axis=1, keepdims=True) + rank1
    dest2 = jnp.sum(oh2 * offs, axis=1, keepdims=True) + rank2
    d0_ref[...] = dest1.astype(jnp.int32)
    d1_ref[...] = dest2.astype(jnp.int32)
    g0_ref[...] = jnp.broadcast_to(v1, (T, 128))
    g1_ref[...] = jnp.broadcast_to(v2, (T, 128))

    ts = lax.broadcasted_iota(jnp.int32, (TMAPN, E), 0).astype(jnp.float32) * float(TILE)
    le = (jnp.broadcast_to(offs, (TMAPN, E)) <= ts).astype(jnp.float32)
    tmap_ref[...] = (jnp.sum(le, axis=1, keepdims=True) - 1.0).astype(jnp.int32)


def _router(xf, wg, rb):
    return pl.pallas_call(
        _router_body,
        out_shape=[
            jax.ShapeDtypeStruct((T, 1), jnp.int32),
            jax.ShapeDtypeStruct((T, 1), jnp.int32),
            jax.ShapeDtypeStruct((T, 128), jnp.float32),
            jax.ShapeDtypeStruct((T, 128), jnp.float32),
            jax.ShapeDtypeStruct((TMAPN, 1), jnp.int32),
            jax.ShapeDtypeStruct((1, 1), jnp.float32),
        ],
    )(xf, wg, rb)


def _dispatch_body(x_hbm, d0_hbm, d1_hbm, xs_hbm,
                   idx0, idx1, xbuf, sem):
    c = lax.axis_index("c")
    s = lax.axis_index("s")
    wid = s * 2 + c
    t0 = wid * TOKW
    pltpu.sync_copy(d0_hbm.at[pl.ds(t0, TOKW)], idx0)
    pltpu.sync_copy(d1_hbm.at[pl.ds(t0, TOKW)], idx1)
    pltpu.sync_copy(x_hbm.at[pl.ds(t0, TOKW)], xbuf)
    cp0 = pltpu.async_copy(xbuf, xs_hbm.at[idx0], sem)
    cp1 = pltpu.async_copy(xbuf, xs_hbm.at[idx1], sem)
    cp0.wait()
    cp1.wait()


def _dispatch(xf, d0, d1):
    mesh = plsc.VectorSubcoreMesh(core_axis_name="c", subcore_axis_name="s")
    fn = functools.partial(
        pl.kernel,
        out_type=jax.ShapeDtypeStruct((NPAD, C), jnp.float32),
        mesh=mesh,
        scratch_types=[
            pltpu.VMEM((TOKW,), jnp.int32),
            pltpu.VMEM((TOKW,), jnp.int32),
            pltpu.VMEM((TOKW, C), jnp.float32),
            pltpu.SemaphoreType.DMA,
        ],
    )(_dispatch_body)
    return fn(xf, d0, d1)


def _ffn_body(tmap_ref, xs_ref, w1a_ref, w1b_ref, b1_ref,
              w2a_ref, w2b_ref, b2_ref, o_ref):
    xb = xs_ref[...]
    dn = (((1,), (0,)), ((), ()))
    ha = lax.dot_general(xb, w1a_ref[0], dn,
                         preferred_element_type=jnp.float32)
    hb = lax.dot_general(xb, w1b_ref[0], dn,
                         preferred_element_type=jnp.float32)
    ha = ha + b1_ref[0][:, 0:H // 2]
    hb = hb + b1_ref[0][:, H // 2:H]
    ha = 0.5 * ha * (1.0 + lax.erf(ha * 0.7071067811865476))
    hb = 0.5 * hb * (1.0 + lax.erf(hb * 0.7071067811865476))
    o = lax.dot_general(ha, w2a_ref[0], dn,
                        preferred_element_type=jnp.float32)
    o = o + lax.dot_general(hb, w2b_ref[0], dn,
                            preferred_element_type=jnp.float32)
    o = o + b2_ref[0]
    o_ref[...] = o


def _ffn(tmap, xs, w1, b1, w2, b2):
    grid_spec = pltpu.PrefetchScalarGridSpec(
        num_scalar_prefetch=1,
        grid=(NTILES,),
        in_specs=[
            pl.BlockSpec((TILE, C), lambda t, tm: (t, 0)),
            pl.BlockSpec((1, C, H // 2), lambda t, tm: (tm[t], 0, 0)),
            pl.BlockSpec((1, C, H // 2), lambda t, tm: (tm[t], 0, 1)),
            pl.BlockSpec((1, 1, H), lambda t, tm: (tm[t], 0, 0)),
            pl.BlockSpec((1, H // 2, C), lambda t, tm: (tm[t], 0, 0)),
            pl.BlockSpec((1, H // 2, C), lambda t, tm: (tm[t], 1, 0)),
            pl.BlockSpec((1, 1, C), lambda t, tm: (tm[t], 0, 0)),
        ],
        out_specs=pl.BlockSpec((TILE, C), lambda t, tm: (t, 0)),
    )
    return pl.pallas_call(
        _ffn_body,
        grid_spec=grid_spec,
        out_shape=jax.ShapeDtypeStruct((NPAD, C), jnp.float32),
        compiler_params=pltpu.CompilerParams(
            dimension_semantics=("arbitrary",)),
    )(tmap, xs, w1, w1, b1, w2, w2, b2)


def _combine_body(os_hbm, d0_hbm, d1_hbm, g0_hbm, g1_hbm, y_hbm,
                  idx0, idx1, b0, b1, gb0, gb1, sem):
    c = lax.axis_index("c")
    s = lax.axis_index("s")
    wid = s * 2 + c
    t0 = wid * TOKW
    for ch in range(TOKW // CH):
        base = t0 + ch * CH
        pltpu.sync_copy(d0_hbm.at[pl.ds(base, CH)], idx0)
        pltpu.sync_copy(d1_hbm.at[pl.ds(base, CH)], idx1)
        cp0 = pltpu.async_copy(os_hbm.at[idx0], b0, sem)
        cp1 = pltpu.async_copy(os_hbm.at[idx1], b1, sem)
        pltpu.sync_copy(g0_hbm.at[pl.ds(base, CH)], gb0)
        pltpu.sync_copy(g1_hbm.at[pl.ds(base, CH)], gb1)
        cp0.wait()
        cp1.wait()

        def body(r, carry):
            gv0 = gb0[r, 0:16]
            gv1 = gb1[r, 0:16]
            for cc in range(C // 16):
                sl = pl.ds(cc * 16, 16)
                b0[r, sl] = b0[r, sl] * gv0 + b1[r, sl] * gv1
            return carry

        lax.fori_loop(0, CH, body, 0)
        pltpu.sync_copy(b0, y_hbm.at[pl.ds(base, CH)])


def _combine(os, d0, d1, g0, g1):
    mesh = plsc.VectorSubcoreMesh(core_axis_name="c", subcore_axis_name="s")
    fn = functools.partial(
        pl.kernel,
        out_type=jax.ShapeDtypeStruct((T, C), jnp.float32),
        mesh=mesh,
        scratch_types=[
            pltpu.VMEM((CH,), jnp.int32),
            pltpu.VMEM((CH,), jnp.int32),
            pltpu.VMEM((CH, C), jnp.float32),
            pltpu.VMEM((CH, C), jnp.float32),
            pltpu.VMEM((CH, 128), jnp.float32),
            pltpu.VMEM((CH, 128), jnp.float32),
            pltpu.SemaphoreType.DMA,
        ],
    )(_combine_body)
    return fn(os, d0, d1, g0, g1)


def kernel(x, w_gating, router_bias, fc1_weight, fc1_bias, fc2_weight,
           fc2_bias):
    xf = x.reshape(T, C)
    rb = router_bias.reshape(1, E)
    d0, d1, g0, g1, tmapN, aux = _router(xf, w_gating, rb)
    d0f = d0.reshape(T)
    d1f = d1.reshape(T)
    tmap = tmapN.reshape(TMAPN)[:NTILES]
    xs = _dispatch(xf, d0f, d1f)
    os_ = _ffn(tmap, xs, fc1_weight, fc1_bias.reshape(E, 1, H),
               fc2_weight, fc2_bias.reshape(E, 1, C))
    y = _combine(os_, d0f, d1f, g0, g1)
    return (y.reshape(1, T, C), aux.reshape(()))


# speedup vs baseline: 1.0331x; 1.0087x over previous
import functools

import jax
import jax.numpy as jnp
from jax import lax
from jax.experimental import pallas as pl
from jax.experimental.pallas import tpu as pltpu
from jax.experimental.pallas import tpu_sc as plsc

T = 2048
C = 1024
E = 16
H = 2048
KTOP = 2
TILE = 128
NTILES = T * KTOP // TILE + E
TMAPN = ((NTILES + 7) // 8) * 8
NPAD = NTILES * TILE
NW = 32
TOKW = T // NW
CH = TOKW // 2


def _router_body(x_ref, wg_ref, rb_ref,
                 d0_ref, d1_ref, g0_ref, g1_ref, tmap_ref, aux_ref):
    x = x_ref[...]
    wg = wg_ref[...]
    logits = lax.dot_general(x, wg, (((1,), (1,)), ((), ())),
                             preferred_element_type=jnp.float32)
    logits = logits + rb_ref[...]
    m = jnp.max(logits, axis=1, keepdims=True)
    p = jnp.exp(logits - m)
    gates = p / jnp.sum(p, axis=1, keepdims=True)

    imp = jnp.sum(gates, axis=0, keepdims=True) * (1.0 / T)
    aux_ref[...] = float(E) * jnp.sum(imp * imp, axis=1, keepdims=True)

    lane = lax.broadcasted_iota(jnp.int32, (T, E), 1).astype(jnp.float32)
    v1 = jnp.max(gates, axis=1, keepdims=True)
    i1 = jnp.min(jnp.where(gates == v1, lane, 1e9), axis=1, keepdims=True)
    masked = jnp.where(lane == i1, -1.0, gates)
    v2 = jnp.max(masked, axis=1, keepdims=True)
    i2 = jnp.min(jnp.where(masked == v2, lane, 1e9), axis=1, keepdims=True)
    oh1 = (lane == i1).astype(jnp.float32)
    oh2 = (lane == i2).astype(jnp.float32)

    r_i = lax.broadcasted_iota(jnp.int32, (T, T), 0)
    c_i = lax.broadcasted_iota(jnp.int32, (T, T), 1)
    tri = (c_i <= r_i).astype(jnp.bfloat16)
    incl1 = lax.dot_general(tri, oh1.astype(jnp.bfloat16),
                            (((1,), (0,)), ((), ())),
                            preferred_element_type=jnp.float32)
    incl2 = lax.dot_general(tri, oh2.astype(jnp.bfloat16),
                            (((1,), (0,)), ((), ())),
                            preferred_element_type=jnp.float32)
    cnt1 = jnp.sum(oh1, axis=0, keepdims=True)
    cnt2 = jnp.sum(oh2, axis=0, keepdims=True)
    rank1 = jnp.sum((incl1 - oh1) * oh1, axis=1, keepdims=True)
    rank2 = jnp.sum((incl2 - oh2 + cnt1) * oh2, axis=1, keepdims=True)

    counts = cnt1 + cnt2
    padded = jnp.floor((counts + float(TILE - 1)) * (1.0 / TILE)) * float(TILE)
    re = lax.broadcasted_iota(jnp.int32, (E, E), 0)
    ce = lax.broadcasted_iota(jnp.int32, (E, E), 1)
    tri_e = (re < ce).astype(jnp.float32)
    offs = lax.dot_general(padded, tri_e, (((1,), (0,)), ((), ())),
                           preferred_element_type=jnp.float32)

    dest1 = jnp.sum(oh1 * offs, axis=1, keepdims=True) + rank1
    dest2 = jnp.sum(oh2 * offs, axis=1, keepdims=True) + rank2
    d0_ref[...] = dest1.astype(jnp.int32)
    d1_ref[...] = dest2.astype(jnp.int32)
    g0_ref[...] = jnp.broadcast_to(v1, (T, 128))
    g1_ref[...] = jnp.broadcast_to(v2, (T, 128))

    ts = lax.broadcasted_iota(jnp.int32, (TMAPN, E), 0).astype(jnp.float32) * float(TILE)
    le = (jnp.broadcast_to(offs, (TMAPN, E)) <= ts).astype(jnp.float32)
    tmap_ref[...] = (jnp.sum(le, axis=1, keepdims=True) - 1.0).astype(jnp.int32)


def _router(xf, wg, rb):
    return pl.pallas_call(
        _router_body,
        out_shape=[
            jax.ShapeDtypeStruct((T, 1), jnp.int32),
            jax.ShapeDtypeStruct((T, 1), jnp.int32),
            jax.ShapeDtypeStruct((T, 128), jnp.float32),
            jax.ShapeDtypeStruct((T, 128), jnp.float32),
            jax.ShapeDtypeStruct((TMAPN, 1), jnp.int32),
            jax.ShapeDtypeStruct((1, 1), jnp.float32),
        ],
    )(xf, wg, rb)


def _dispatch_body(x_hbm, d0_hbm, d1_hbm, xs_hbm,
                   idx0, idx1, xbuf, sem):
    c = lax.axis_index("c")
    s = lax.axis_index("s")
    wid = s * 2 + c
    t0 = wid * TOKW
    pltpu.sync_copy(d0_hbm.at[pl.ds(t0, TOKW)], idx0)
    pltpu.sync_copy(d1_hbm.at[pl.ds(t0, TOKW)], idx1)
    pltpu.sync_copy(x_hbm.at[pl.ds(t0, TOKW)], xbuf)
    cp0 = pltpu.async_copy(xbuf, xs_hbm.at[idx0], sem)
    cp1 = pltpu.async_copy(xbuf, xs_hbm.at[idx1], sem)
    cp0.wait()
    cp1.wait()


def _dispatch(xf, d0, d1):
    mesh = plsc.VectorSubcoreMesh(core_axis_name="c", subcore_axis_name="s")
    fn = functools.partial(
        pl.kernel,
        out_type=jax.ShapeDtypeStruct((NPAD, C), jnp.float32),
        mesh=mesh,
        scratch_types=[
            pltpu.VMEM((TOKW,), jnp.int32),
            pltpu.VMEM((TOKW,), jnp.int32),
            pltpu.VMEM((TOKW, C), jnp.float32),
            pltpu.SemaphoreType.DMA,
        ],
    )(_dispatch_body)
    return fn(xf, d0, d1)


def _ffn_body(tmap_ref, xs_ref, w1_ref, b1_ref, w2_ref, b2_ref, o_ref):
    xb = xs_ref[...]
    h = lax.dot_general(xb, w1_ref[0], (((1,), (0,)), ((), ())),
                        preferred_element_type=jnp.float32)
    h = h + b1_ref[0]
    h = 0.5 * h * (1.0 + lax.erf(h * 0.7071067811865476))
    o = lax.dot_general(h, w2_ref[0], (((1,), (0,)), ((), ())),
                        preferred_element_type=jnp.float32)
    o = o + b2_ref[0]
    o_ref[...] = o


def _ffn(tmap, xs, w1, b1, w2, b2):
    grid_spec = pltpu.PrefetchScalarGridSpec(
        num_scalar_prefetch=1,
        grid=(NTILES,),
        in_specs=[
            pl.BlockSpec((TILE, C), lambda t, tm: (t, 0)),
            pl.BlockSpec((1, C, H), lambda t, tm: (tm[t], 0, 0)),
            pl.BlockSpec((1, 1, H), lambda t, tm: (tm[t], 0, 0)),
            pl.BlockSpec((1, H, C), lambda t, tm: (tm[t], 0, 0)),
            pl.BlockSpec((1, 1, C), lambda t, tm: (tm[t], 0, 0)),
        ],
        out_specs=pl.BlockSpec((TILE, C), lambda t, tm: (t, 0)),
    )
    return pl.pallas_call(
        _ffn_body,
        grid_spec=grid_spec,
        out_shape=jax.ShapeDtypeStruct((NPAD, C), jnp.float32),
        compiler_params=pltpu.CompilerParams(
            dimension_semantics=("arbitrary",)),
    )(tmap, xs, w1, b1, w2, b2)


def _combine_body(os_hbm, d0_hbm, d1_hbm, g0_hbm, g1_hbm, y_hbm,
                  idx0, idx1, b0, b1, gb0, gb1, sem):
    c = lax.axis_index("c")
    s = lax.axis_index("s")
    wid = s * 2 + c
    t0 = wid * TOKW
    for ch in range(TOKW // CH):
        base = t0 + ch * CH
        pltpu.sync_copy(d0_hbm.at[pl.ds(base, CH)], idx0)
        pltpu.sync_copy(d1_hbm.at[pl.ds(base, CH)], idx1)
        cp0 = pltpu.async_copy(os_hbm.at[idx0], b0, sem)
        cp1 = pltpu.async_copy(os_hbm.at[idx1], b1, sem)
        pltpu.sync_copy(g0_hbm.at[pl.ds(base, CH)], gb0)
        pltpu.sync_copy(g1_hbm.at[pl.ds(base, CH)], gb1)
        cp0.wait()
        cp1.wait()

        def body(r, carry):
            gv0 = gb0[r, 0:16]
            gv1 = gb1[r, 0:16]
            for cc in range(C // 16):
                sl = pl.ds(cc * 16, 16)
                b0[r, sl] = b0[r, sl] * gv0 + b1[r, sl] * gv1
            return carry

        lax.fori_loop(0, CH, body, 0)
        pltpu.sync_copy(b0, y_hbm.at[pl.ds(base, CH)])


def _combine(os, d0, d1, g0, g1):
    mesh = plsc.VectorSubcoreMesh(core_axis_name="c", subcore_axis_name="s")
    fn = functools.partial(
        pl.kernel,
        out_type=jax.ShapeDtypeStruct((T, C), jnp.float32),
        mesh=mesh,
        scratch_types=[
            pltpu.VMEM((CH,), jnp.int32),
            pltpu.VMEM((CH,), jnp.int32),
            pltpu.VMEM((CH, C), jnp.float32),
            pltpu.VMEM((CH, C), jnp.float32),
            pltpu.VMEM((CH, 128), jnp.float32),
            pltpu.VMEM((CH, 128), jnp.float32),
            pltpu.SemaphoreType.DMA,
        ],
    )(_combine_body)
    return fn(os, d0, d1, g0, g1)


def kernel(x, w_gating, router_bias, fc1_weight, fc1_bias, fc2_weight,
           fc2_bias):
    xf = x.reshape(T, C)
    rb = router_bias.reshape(1, E)
    d0, d1, g0, g1, tmapN, aux = _router(xf, w_gating, rb)
    d0f = d0.reshape(T)
    d1f = d1.reshape(T)
    tmap = tmapN.reshape(TMAPN)[:NTILES]
    xs = _dispatch(xf, d0f, d1f)
    os_ = _ffn(tmap, xs, fc1_weight, fc1_bias.reshape(E, 1, H),
               fc2_weight, fc2_bias.reshape(E, 1, C))
    y = _combine(os_, d0f, d1f, g0, g1)
    return (y.reshape(1, T, C), aux.reshape(()))


# speedup vs baseline: 1.0498x; 1.0161x over previous
import functools

import jax
import jax.numpy as jnp
from jax import lax
from jax.experimental import pallas as pl
from jax.experimental.pallas import tpu as pltpu
from jax.experimental.pallas import tpu_sc as plsc

T = 2048
C = 1024
E = 16
H = 2048
KTOP = 2
TILE = 128
NTILES = T * KTOP // TILE + E
TMAPN = ((NTILES + 7) // 8) * 8
NPAD = NTILES * TILE
NW = 32
TOKW = T // NW
CH = TOKW // 2


def _router_body(x_ref, wg_ref, rb_ref,
                 d0_ref, d1_ref, g0_ref, g1_ref, tmap_ref, aux_ref):
    x = x_ref[...]
    wg = wg_ref[...]
    logits = lax.dot_general(x, wg, (((1,), (1,)), ((), ())),
                             preferred_element_type=jnp.float32)
    logits = logits + rb_ref[...]
    m = jnp.max(logits, axis=1, keepdims=True)
    p = jnp.exp(logits - m)
    gates = p / jnp.sum(p, axis=1, keepdims=True)

    imp = jnp.sum(gates, axis=0, keepdims=True) * (1.0 / T)
    aux_ref[...] = float(E) * jnp.sum(imp * imp, axis=1, keepdims=True)

    lane = lax.broadcasted_iota(jnp.int32, (T, E), 1).astype(jnp.float32)
    v1 = jnp.max(gates, axis=1, keepdims=True)
    i1 = jnp.min(jnp.where(gates == v1, lane, 1e9), axis=1, keepdims=True)
    masked = jnp.where(lane == i1, -1.0, gates)
    v2 = jnp.max(masked, axis=1, keepdims=True)
    i2 = jnp.min(jnp.where(masked == v2, lane, 1e9), axis=1, keepdims=True)
    oh1 = (lane == i1).astype(jnp.float32)
    oh2 = (lane == i2).astype(jnp.float32)

    r_i = lax.broadcasted_iota(jnp.int32, (T, T), 0)
    c_i = lax.broadcasted_iota(jnp.int32, (T, T), 1)
    tri = (c_i <= r_i).astype(jnp.bfloat16)
    incl1 = lax.dot_general(tri, oh1.astype(jnp.bfloat16),
                            (((1,), (0,)), ((), ())),
                            preferred_element_type=jnp.float32)
    incl2 = lax.dot_general(tri, oh2.astype(jnp.bfloat16),
                            (((1,), (0,)), ((), ())),
                            preferred_element_type=jnp.float32)
    cnt1 = jnp.sum(oh1, axis=0, keepdims=True)
    cnt2 = jnp.sum(oh2, axis=0, keepdims=True)
    rank1 = jnp.sum((incl1 - oh1) * oh1, axis=1)
    rank2 = jnp.sum((incl2 - oh2 + cnt1) * oh2, axis=1)

    counts = cnt1 + cnt2
    padded = jnp.floor((counts + float(TILE - 1)) * (1.0 / TILE)) * float(TILE)
    re = lax.broadcasted_iota(jnp.int32, (E, E), 0)
    ce = lax.broadcasted_iota(jnp.int32, (E, E), 1)
    tri_e = (re < ce).astype(jnp.float32)
    offs = lax.dot_general(padded, tri_e, (((1,), (0,)), ((), ())),
                           preferred_element_type=jnp.float32)

    dest1 = jnp.sum(oh1 * offs, axis=1) + rank1
    dest2 = jnp.sum(oh2 * offs, axis=1) + rank2
    d0_ref[...] = dest1.astype(jnp.int32)
    d1_ref[...] = dest2.astype(jnp.int32)
    g0_ref[...] = jnp.broadcast_to(v1, (T, 128))
    g1_ref[...] = jnp.broadcast_to(v2, (T, 128))

    ts = lax.broadcasted_iota(jnp.int32, (TMAPN, E), 0).astype(jnp.float32) * float(TILE)
    le = (jnp.broadcast_to(offs, (TMAPN, E)) <= ts).astype(jnp.float32)
    tmap_ref[...] = (jnp.sum(le, axis=1) - 1.0).astype(jnp.int32)


def _router(xf, wg, rb):
    return pl.pallas_call(
        _router_body,
        out_shape=[
            jax.ShapeDtypeStruct((T,), jnp.int32),
            jax.ShapeDtypeStruct((T,), jnp.int32),
            jax.ShapeDtypeStruct((T, 128), jnp.float32),
            jax.ShapeDtypeStruct((T, 128), jnp.float32),
            jax.ShapeDtypeStruct((TMAPN,), jnp.int32),
            jax.ShapeDtypeStruct((1, 1), jnp.float32),
        ],
    )(xf, wg, rb)


def _dispatch_body(x_hbm, d0_hbm, d1_hbm, xs_hbm,
                   idx0, idx1, xbuf, sem):
    c = lax.axis_index("c")
    s = lax.axis_index("s")
    wid = s * 2 + c
    t0 = wid * TOKW
    pltpu.sync_copy(d0_hbm.at[pl.ds(t0, TOKW)], idx0)
    pltpu.sync_copy(d1_hbm.at[pl.ds(t0, TOKW)], idx1)
    pltpu.sync_copy(x_hbm.at[pl.ds(t0, TOKW)], xbuf)
    cp0 = pltpu.async_copy(xbuf, xs_hbm.at[idx0], sem)
    cp1 = pltpu.async_copy(xbuf, xs_hbm.at[idx1], sem)
    cp0.wait()
    cp1.wait()


def _dispatch(xf, d0, d1):
    mesh = plsc.VectorSubcoreMesh(core_axis_name="c", subcore_axis_name="s")
    fn = functools.partial(
        pl.kernel,
        out_type=jax.ShapeDtypeStruct((NPAD, C), jnp.float32),
        mesh=mesh,
        scratch_types=[
            pltpu.VMEM((TOKW,), jnp.int32),
            pltpu.VMEM((TOKW,), jnp.int32),
            pltpu.VMEM((TOKW, C), jnp.float32),
            pltpu.SemaphoreType.DMA,
        ],
    )(_dispatch_body)
    return fn(xf, d0, d1)


def _ffn_body(tmap_ref, xs_ref, w1_ref, b1_ref, w2_ref, b2_ref, o_ref):
    xb = xs_ref[...]
    h = lax.dot_general(xb, w1_ref[0], (((1,), (0,)), ((), ())),
                        preferred_element_type=jnp.float32)
    h = h + b1_ref[0]
    h = 0.5 * h * (1.0 + lax.erf(h * 0.7071067811865476))
    o = lax.dot_general(h, w2_ref[0], (((1,), (0,)), ((), ())),
                        preferred_element_type=jnp.float32)
    o = o + b2_ref[0]
    o_ref[...] = o


def _ffn(tmap, xs, w1, b1, w2, b2):
    grid_spec = pltpu.PrefetchScalarGridSpec(
        num_scalar_prefetch=1,
        grid=(NTILES,),
        in_specs=[
            pl.BlockSpec((TILE, C), lambda t, tm: (t, 0)),
            pl.BlockSpec((1, C, H), lambda t, tm: (tm[t], 0, 0)),
            pl.BlockSpec((1, 1, H), lambda t, tm: (tm[t], 0, 0)),
            pl.BlockSpec((1, H, C), lambda t, tm: (tm[t], 0, 0)),
            pl.BlockSpec((1, 1, C), lambda t, tm: (tm[t], 0, 0)),
        ],
        out_specs=pl.BlockSpec((TILE, C), lambda t, tm: (t, 0)),
    )
    return pl.pallas_call(
        _ffn_body,
        grid_spec=grid_spec,
        out_shape=jax.ShapeDtypeStruct((NPAD, C), jnp.float32),
        compiler_params=pltpu.CompilerParams(
            dimension_semantics=("arbitrary",)),
    )(tmap, xs, w1, b1, w2, b2)


def _combine_body(os_hbm, d0_hbm, d1_hbm, g0_hbm, g1_hbm, y_hbm,
                  idx0, idx1, b0, b1, gb0, gb1, sem):
    c = lax.axis_index("c")
    s = lax.axis_index("s")
    wid = s * 2 + c
    t0 = wid * TOKW
    for ch in range(TOKW // CH):
        base = t0 + ch * CH
        pltpu.sync_copy(d0_hbm.at[pl.ds(base, CH)], idx0)
        pltpu.sync_copy(d1_hbm.at[pl.ds(base, CH)], idx1)
        cp0 = pltpu.async_copy(os_hbm.at[idx0], b0, sem)
        cp1 = pltpu.async_copy(os_hbm.at[idx1], b1, sem)
        pltpu.sync_copy(g0_hbm.at[pl.ds(base, CH)], gb0)
        pltpu.sync_copy(g1_hbm.at[pl.ds(base, CH)], gb1)
        cp0.wait()
        cp1.wait()

        def body(r, carry):
            gv0 = gb0[r, 0:16]
            gv1 = gb1[r, 0:16]
            for cc in range(C // 16):
                sl = pl.ds(cc * 16, 16)
                b0[r, sl] = b0[r, sl] * gv0 + b1[r, sl] * gv1
            return carry

        lax.fori_loop(0, CH, body, 0)
        pltpu.sync_copy(b0, y_hbm.at[pl.ds(base, CH)])


def _combine(os, d0, d1, g0, g1):
    mesh = plsc.VectorSubcoreMesh(core_axis_name="c", subcore_axis_name="s")
    fn = functools.partial(
        pl.kernel,
        out_type=jax.ShapeDtypeStruct((T, C), jnp.float32),
        mesh=mesh,
        scratch_types=[
            pltpu.VMEM((CH,), jnp.int32),
            pltpu.VMEM((CH,), jnp.int32),
            pltpu.VMEM((CH, C), jnp.float32),
            pltpu.VMEM((CH, C), jnp.float32),
            pltpu.VMEM((CH, 128), jnp.float32),
            pltpu.VMEM((CH, 128), jnp.float32),
            pltpu.SemaphoreType.DMA,
        ],
    )(_combine_body)
    return fn(os, d0, d1, g0, g1)


def kernel(x, w_gating, router_bias, fc1_weight, fc1_bias, fc2_weight,
           fc2_bias):
    xf = x.reshape(T, C)
    rb = router_bias.reshape(1, E)
    d0f, d1f, g0, g1, tmapN, aux = _router(xf, w_gating, rb)
    tmap = tmapN[:NTILES]
    xs = _dispatch(xf, d0f, d1f)
    os_ = _ffn(tmap, xs, fc1_weight, fc1_bias.reshape(E, 1, H),
               fc2_weight, fc2_bias.reshape(E, 1, C))
    y = _combine(os_, d0f, d1f, g0, g1)
    return (y.reshape(1, T, C), aux.reshape(()))
